# baseline (speedup 1.0000x reference)
import jax
import jax.numpy as jnp
from jax.experimental import pallas as pl
from jax.experimental.pallas import tpu as pltpu

B = 32
L = 2048
D_IN = 128
EMBED_DIM = 128
BLK = 32768


def _fused_kernel(x_ref, w_ref, b_ref, pos_ref, out_ref):
    acc = jnp.dot(x_ref[...], w_ref[...], preferred_element_type=jnp.float32)
    m = BLK // L
    acc = acc.reshape(m, L, EMBED_DIM) + pos_ref[...][None, :, :] + b_ref[...]
    out_ref[...] = acc.reshape(BLK, EMBED_DIM)


def kernel(x, W, b, pos_table):
    x2 = x.reshape(B * L, D_IN)
    b2 = b.reshape(1, EMBED_DIM)
    assert (B * L) % BLK == 0 and BLK % L == 0
    grid = (B * L) // BLK
    out = pl.pallas_call(
        _fused_kernel,
        grid=(grid,),
        in_specs=[
            pl.BlockSpec((BLK, D_IN), lambda i: (i, 0),
                         pipeline_mode=pl.Buffered(buffer_count=1)),
            pl.BlockSpec((D_IN, EMBED_DIM), lambda i: (0, 0)),
            pl.BlockSpec((1, EMBED_DIM), lambda i: (0, 0)),
            pl.BlockSpec((L, EMBED_DIM), lambda i: (0, 0)),
        ],
        out_specs=pl.BlockSpec((BLK, EMBED_DIM), lambda i: (i, 0)),
        out_shape=jax.ShapeDtypeStruct((B * L, EMBED_DIM), jnp.float32),
        compiler_params=pltpu.CompilerParams(
            dimension_semantics=("parallel",),
        ),
    )(x2, W, b2, pos_table)
    return out.reshape(B, L, EMBED_DIM)


# speedup vs baseline: 1.2398x; 1.2398x over previous
import jax
import jax.numpy as jnp
from jax.experimental import pallas as pl
from jax.experimental.pallas import tpu as pltpu

B = 32
L = 2048
D_IN = 128
EMBED_DIM = 128
CHUNK = 2048
NBUF = 6
NCHUNK = (B * L) // CHUNK


def _fused_kernel(x_hbm, w_ref, b_ref, pos_ref, out_hbm,
                  in_buf, out_buf, in_sems, out_sems):
    def load(i):
        pltpu.make_async_copy(
            x_hbm.at[pl.ds(i * CHUNK, CHUNK), :],
            in_buf.at[i % NBUF],
            in_sems.at[i % NBUF],
        ).start()

    def store(i):
        pltpu.make_async_copy(
            out_buf.at[i % NBUF],
            out_hbm.at[pl.ds(i * CHUNK, CHUNK), :],
            out_sems.at[i % NBUF],
        ).start()

    for j in range(min(NBUF, NCHUNK)):
        load(j)

    pos_bias = pos_ref[...] + b_ref[...]
    for i in range(NCHUNK):
        slot = i % NBUF
        pltpu.make_async_copy(
            x_hbm.at[pl.ds(i * CHUNK, CHUNK), :],
            in_buf.at[slot],
            in_sems.at[slot],
        ).wait()
        if i >= NBUF:
            pltpu.make_async_copy(
                out_buf.at[slot],
                out_hbm.at[pl.ds((i - NBUF) * CHUNK, CHUNK), :],
                out_sems.at[slot],
            ).wait()
        acc = jnp.dot(in_buf[slot], w_ref[...],
                      preferred_element_type=jnp.float32)
        out_buf[slot] = acc + pos_bias
        store(i)
        nxt = i + NBUF
        if nxt < NCHUNK:
            load(nxt)

    for i in range(max(0, NCHUNK - NBUF), NCHUNK):
        slot = i % NBUF
        pltpu.make_async_copy(
            out_buf.at[slot],
            out_hbm.at[pl.ds(i * CHUNK, CHUNK), :],
            out_sems.at[slot],
        ).wait()


def kernel(x, W, b, pos_table):
    x2 = x.reshape(B * L, D_IN)
    b2 = b.reshape(1, EMBED_DIM)
    out = pl.pallas_call(
        _fused_kernel,
        in_specs=[
            pl.BlockSpec(memory_space=pltpu.MemorySpace.HBM),
            pl.BlockSpec(memory_space=pltpu.MemorySpace.VMEM),
            pl.BlockSpec(memory_space=pltpu.MemorySpace.VMEM),
            pl.BlockSpec(memory_space=pltpu.MemorySpace.VMEM),
        ],
        out_specs=pl.BlockSpec(memory_space=pltpu.MemorySpace.HBM),
        out_shape=jax.ShapeDtypeStruct((B * L, EMBED_DIM), jnp.float32),
        scratch_shapes=[
            pltpu.MemorySpace.VMEM((NBUF, CHUNK, D_IN), jnp.float32),
            pltpu.MemorySpace.VMEM((NBUF, CHUNK, EMBED_DIM), jnp.float32),
            pltpu.SemaphoreType.DMA((NBUF,)),
            pltpu.SemaphoreType.DMA((NBUF,)),
        ],
    )(x2, W, b2, pos_table)
    return out.reshape(B, L, EMBED_DIM)


# speedup vs baseline: 1.2444x; 1.0037x over previous
import jax
import jax.numpy as jnp
from jax.experimental import pallas as pl
from jax.experimental.pallas import tpu as pltpu

B = 32
L = 2048
D_IN = 128
EMBED_DIM = 128
CHUNK = 4096
NBUF = 4
NCHUNK = (B * L) // CHUNK


def _fused_kernel(x_hbm, w_ref, b_ref, pos_ref, out_hbm,
                  in_buf, out_buf, in_sems, out_sems):
    def load(i):
        pltpu.make_async_copy(
            x_hbm.at[pl.ds(i * CHUNK, CHUNK), :],
            in_buf.at[i % NBUF],
            in_sems.at[i % NBUF],
        ).start()

    def store(i):
        pltpu.make_async_copy(
            out_buf.at[i % NBUF],
            out_hbm.at[pl.ds(i * CHUNK, CHUNK), :],
            out_sems.at[i % NBUF],
        ).start()

    for j in range(min(NBUF, NCHUNK)):
        load(j)

    pos_bias = pos_ref[...] + b_ref[...]
    for i in range(NCHUNK):
        slot = i % NBUF
        pltpu.make_async_copy(
            x_hbm.at[pl.ds(i * CHUNK, CHUNK), :],
            in_buf.at[slot],
            in_sems.at[slot],
        ).wait()
        if i >= NBUF:
            pltpu.make_async_copy(
                out_buf.at[slot],
                out_hbm.at[pl.ds((i - NBUF) * CHUNK, CHUNK), :],
                out_sems.at[slot],
            ).wait()
        acc = jnp.dot(in_buf[slot], w_ref[...],
                      preferred_element_type=jnp.float32)
        if CHUNK == L:
            out_buf[slot] = acc + pos_bias
        else:
            acc = acc.reshape(CHUNK // L, L, EMBED_DIM) + pos_bias[None, :, :]
            out_buf[slot] = acc.reshape(CHUNK, EMBED_DIM)
        store(i)
        nxt = i + NBUF
        if nxt < NCHUNK:
            load(nxt)

    for i in range(max(0, NCHUNK - NBUF), NCHUNK):
        slot = i % NBUF
        pltpu.make_async_copy(
            out_buf.at[slot],
            out_hbm.at[pl.ds(i * CHUNK, CHUNK), :],
            out_sems.at[slot],
        ).wait()


def kernel(x, W, b, pos_table):
    x2 = x.reshape(B * L, D_IN)
    b2 = b.reshape(1, EMBED_DIM)
    out = pl.pallas_call(
        _fused_kernel,
        in_specs=[
            pl.BlockSpec(memory_space=pltpu.MemorySpace.HBM),
            pl.BlockSpec(memory_space=pltpu.MemorySpace.VMEM),
            pl.BlockSpec(memory_space=pltpu.MemorySpace.VMEM),
            pl.BlockSpec(memory_space=pltpu.MemorySpace.VMEM),
        ],
        out_specs=pl.BlockSpec(memory_space=pltpu.MemorySpace.HBM),
        out_shape=jax.ShapeDtypeStruct((B * L, EMBED_DIM), jnp.float32),
        scratch_shapes=[
            pltpu.MemorySpace.VMEM((NBUF, CHUNK, D_IN), jnp.float32),
            pltpu.MemorySpace.VMEM((NBUF, CHUNK, EMBED_DIM), jnp.float32),
            pltpu.SemaphoreType.DMA((NBUF,)),
            pltpu.SemaphoreType.DMA((NBUF,)),
        ],
    )(x2, W, b2, pos_table)
    return out.reshape(B, L, EMBED_DIM)


# speedup vs baseline: 1.2504x; 1.0048x over previous
import jax
import jax.numpy as jnp
from jax.experimental import pallas as pl

B = 32
L = 2048
D_IN = 128
EMBED_DIM = 128
BLK = 16384


def _fused_kernel(x_ref, w_ref, b_ref, pos_ref, out_ref):
    acc = jnp.dot(x_ref[...], w_ref[...], preferred_element_type=jnp.float32)
    m = BLK // L
    acc = acc.reshape(m, L, EMBED_DIM) + pos_ref[...][None, :, :] + b_ref[...]
    out_ref[...] = acc.reshape(BLK, EMBED_DIM)


def kernel(x, W, b, pos_table):
    x2 = x.reshape(B * L, D_IN)
    b2 = b.reshape(1, EMBED_DIM)
    assert (B * L) % BLK == 0 and BLK % L == 0
    grid = (B * L) // BLK
    out = pl.pallas_call(
        _fused_kernel,
        grid=(grid,),
        in_specs=[
            pl.BlockSpec((BLK, D_IN), lambda i: (i, 0)),
            pl.BlockSpec((D_IN, EMBED_DIM), lambda i: (0, 0)),
            pl.BlockSpec((1, EMBED_DIM), lambda i: (0, 0)),
            pl.BlockSpec((L, EMBED_DIM), lambda i: (0, 0)),
        ],
        out_specs=pl.BlockSpec((BLK, EMBED_DIM), lambda i: (i, 0)),
        out_shape=jax.ShapeDtypeStruct((B * L, EMBED_DIM), jnp.float32),
    )(x2, W, b2, pos_table)
    return out.reshape(B, L, EMBED_DIM)
